# baseline (speedup 1.0000x reference)
import jax
import jax.numpy as jnp
from jax import lax
from jax.experimental import pallas as pl
from jax.experimental.pallas import tpu as pltpu

N = 1024
D = 64
IDX = 64
H = 8
L = 3
BR = 128


def _msg_kernel(scal_ref, adj_ref, adjd_ref, emb_ref, embT_ref, wiT_ref, wj_ref,
                a_ref, ad_ref):
    for m in range(2):
        A = adj_ref[...] if m == 0 else adjd_ref[...]
        out_ref = a_ref if m == 0 else ad_ref
        base = m * 97
        rw = jnp.dot(emb_ref[...], wiT_ref[m], preferred_element_type=jnp.float32)
        clT = jnp.dot(wj_ref[m], embT_ref[...], preferred_element_type=jnp.float32)
        h1 = []
        for k in range(H):
            wa_k = scal_ref[base + k]
            b1_k = scal_ref[base + H + k]
            h1.append(jnp.maximum(A * wa_k + rw[:, k:k + 1] + (clT[k:k + 1, :] + b1_k), 0.0))
        h2 = []
        for k2 in range(H):
            acc = h1[0] * scal_ref[base + 16 + k2 * H]
            for j in range(1, H):
                acc = acc + h1[j] * scal_ref[base + 16 + k2 * H + j]
            h2.append(jnp.maximum(acc + scal_ref[base + 80 + k2], 0.0))
        out = h2[0] * scal_ref[base + 88]
        for j in range(1, H):
            out = out + h2[j] * scal_ref[base + 88 + j]
        out_ref[...] = out + scal_ref[base + 96]


def _gnn_kernel(a_ref, ad_ref, y_ref, tg_ref, W_ref, b_ref, o_ref):
    x = y_ref[...]
    dn_t = (((0,), (0,)), ((), ()))
    for l in range(L):
        p0 = jnp.dot(a_ref[...], x, preferred_element_type=jnp.float32)
        p1 = lax.dot_general(a_ref[...], x, dn_t, preferred_element_type=jnp.float32)
        p2 = jnp.dot(ad_ref[...], x, preferred_element_type=jnp.float32)
        p3 = lax.dot_general(ad_ref[...], x, dn_t, preferred_element_type=jnp.float32)
        x = (jnp.dot(p0, W_ref[l, 0], preferred_element_type=jnp.float32)
             + jnp.dot(p1, W_ref[l, 1], preferred_element_type=jnp.float32)
             + jnp.dot(p2, W_ref[l, 2], preferred_element_type=jnp.float32)
             + jnp.dot(p3, W_ref[l, 3], preferred_element_type=jnp.float32)
             + jnp.dot(x, W_ref[l, 4], preferred_element_type=jnp.float32)
             + b_ref[l])
        if l < L - 1:
            x = jnp.maximum(x, 0.0)
    ones = jnp.ones((N, D), jnp.float32)
    tgm = lax.dot_general(tg_ref[...], ones, dn_t, preferred_element_type=jnp.float32)
    o_ref[...] = x * (tgm * (1.0 / N))


def kernel(y, adj, adj_deriv, t_grad, idx_emb, msg_W1, msg_b1, msg_W2, msg_b2,
           msg_W3, msg_b3, gnn_W, gnn_b):
    parts = []
    for m in range(2):
        parts += [msg_W1[m, :, 0], msg_b1[m], msg_W2[m].reshape(-1), msg_b2[m],
                  msg_W3[m, 0], msg_b3[m]]
    scal = jnp.concatenate(parts)
    wiT = msg_W1[:, :, 1:1 + IDX].transpose(0, 2, 1)
    wj = msg_W1[:, :, 1 + IDX:]
    embT = idx_emb.T

    nt = N // BR
    a, ad = pl.pallas_call(
        _msg_kernel,
        grid=(nt,),
        in_specs=[
            pl.BlockSpec(memory_space=pltpu.SMEM),
            pl.BlockSpec((BR, N), lambda i: (i, 0)),
            pl.BlockSpec((BR, N), lambda i: (i, 0)),
            pl.BlockSpec((BR, IDX), lambda i: (i, 0)),
            pl.BlockSpec((IDX, N), lambda i: (0, 0)),
            pl.BlockSpec((2, IDX, H), lambda i: (0, 0, 0)),
            pl.BlockSpec((2, H, IDX), lambda i: (0, 0, 0)),
        ],
        out_specs=[
            pl.BlockSpec((BR, N), lambda i: (i, 0)),
            pl.BlockSpec((BR, N), lambda i: (i, 0)),
        ],
        out_shape=[
            jax.ShapeDtypeStruct((N, N), jnp.float32),
            jax.ShapeDtypeStruct((N, N), jnp.float32),
        ],
        compiler_params=pltpu.CompilerParams(
            dimension_semantics=("parallel",),
        ),
        name="msg_mlp",
    )(scal, adj, adj_deriv, idx_emb, embT, wiT, wj)

    out = pl.pallas_call(
        _gnn_kernel,
        in_specs=[
            pl.BlockSpec((N, N), lambda: (0, 0)),
            pl.BlockSpec((N, N), lambda: (0, 0)),
            pl.BlockSpec((N, D), lambda: (0, 0)),
            pl.BlockSpec((N, N), lambda: (0, 0)),
            pl.BlockSpec((L, 5, D, D), lambda: (0, 0, 0, 0)),
            pl.BlockSpec((L, 1, D), lambda: (0, 0, 0)),
        ],
        out_specs=pl.BlockSpec((N, D), lambda: (0, 0)),
        out_shape=jax.ShapeDtypeStruct((N, D), jnp.float32),
        name="gnn_layers",
    )(a, ad, y, t_grad, gnn_W, gnn_b.reshape(L, 1, D))
    return out


# speedup vs baseline: 1.5897x; 1.5897x over previous
import jax
import jax.numpy as jnp
from jax import lax
from jax.experimental import pallas as pl
from jax.experimental.pallas import tpu as pltpu

N = 1024
D = 64
IDX = 64
H = 8
L = 3
BR = 128


def _msg_kernel(scal_ref, adj_ref, adjd_ref, emb_ref, embT_ref, wiT_ref, wj_ref,
                a_ref, ad_ref):
    bf = jnp.bfloat16
    for m in range(2):
        A = (adj_ref[...] if m == 0 else adjd_ref[...]).astype(bf)
        out_ref = a_ref if m == 0 else ad_ref
        base = m * 97
        rw = jnp.dot(emb_ref[...], wiT_ref[m], preferred_element_type=jnp.float32)
        clT = jnp.dot(wj_ref[m], embT_ref[...], preferred_element_type=jnp.float32)
        b1col = jnp.stack([scal_ref[base + H + k] for k in range(H)]).reshape(H, 1)
        rwb = rw.astype(bf)
        clT2 = (clT + b1col).astype(bf)
        h1 = []
        for k in range(H):
            wa_k = scal_ref[base + k].astype(bf)
            h1.append(jnp.maximum(A * wa_k + rwb[:, k:k + 1] + clT2[k:k + 1, :],
                                  bf(0.0)))
        h2 = []
        for k2 in range(H):
            acc = h1[0] * scal_ref[base + 16 + k2 * H].astype(bf)
            for j in range(1, H):
                acc = acc + h1[j] * scal_ref[base + 16 + k2 * H + j].astype(bf)
            h2.append(jnp.maximum(acc + scal_ref[base + 80 + k2].astype(bf), bf(0.0)))
        out = h2[0] * scal_ref[base + 88].astype(bf)
        for j in range(1, H):
            out = out + h2[j] * scal_ref[base + 88 + j].astype(bf)
        out_ref[...] = out + scal_ref[base + 96].astype(bf)


def _gnn_kernel(a_ref, ad_ref, y_ref, tg_ref, W_ref, b_ref, o_ref):
    x = y_ref[...]
    dn_t = (((0,), (0,)), ((), ()))
    for l in range(L):
        xb = x.astype(jnp.bfloat16)
        p0 = jnp.dot(a_ref[...], xb, preferred_element_type=jnp.float32)
        p1 = lax.dot_general(a_ref[...], xb, dn_t, preferred_element_type=jnp.float32)
        p2 = jnp.dot(ad_ref[...], xb, preferred_element_type=jnp.float32)
        p3 = lax.dot_general(ad_ref[...], xb, dn_t, preferred_element_type=jnp.float32)
        x = (jnp.dot(p0, W_ref[l, 0], preferred_element_type=jnp.float32)
             + jnp.dot(p1, W_ref[l, 1], preferred_element_type=jnp.float32)
             + jnp.dot(p2, W_ref[l, 2], preferred_element_type=jnp.float32)
             + jnp.dot(p3, W_ref[l, 3], preferred_element_type=jnp.float32)
             + jnp.dot(x, W_ref[l, 4], preferred_element_type=jnp.float32)
             + b_ref[l])
        if l < L - 1:
            x = jnp.maximum(x, 0.0)
    ones = jnp.ones((N, D), jnp.float32)
    tgm = lax.dot_general(tg_ref[...], ones, dn_t, preferred_element_type=jnp.float32)
    o_ref[...] = x * (tgm * (1.0 / N))


def kernel(y, adj, adj_deriv, t_grad, idx_emb, msg_W1, msg_b1, msg_W2, msg_b2,
           msg_W3, msg_b3, gnn_W, gnn_b):
    parts = []
    for m in range(2):
        parts += [msg_W1[m, :, 0], msg_b1[m], msg_W2[m].reshape(-1), msg_b2[m],
                  msg_W3[m, 0], msg_b3[m]]
    scal = jnp.concatenate(parts)
    wiT = msg_W1[:, :, 1:1 + IDX].transpose(0, 2, 1)
    wj = msg_W1[:, :, 1 + IDX:]
    embT = idx_emb.T

    nt = N // BR
    a, ad = pl.pallas_call(
        _msg_kernel,
        grid=(nt,),
        in_specs=[
            pl.BlockSpec(memory_space=pltpu.SMEM),
            pl.BlockSpec((BR, N), lambda i: (i, 0)),
            pl.BlockSpec((BR, N), lambda i: (i, 0)),
            pl.BlockSpec((BR, IDX), lambda i: (i, 0)),
            pl.BlockSpec((IDX, N), lambda i: (0, 0)),
            pl.BlockSpec((2, IDX, H), lambda i: (0, 0, 0)),
            pl.BlockSpec((2, H, IDX), lambda i: (0, 0, 0)),
        ],
        out_specs=[
            pl.BlockSpec((BR, N), lambda i: (i, 0)),
            pl.BlockSpec((BR, N), lambda i: (i, 0)),
        ],
        out_shape=[
            jax.ShapeDtypeStruct((N, N), jnp.bfloat16),
            jax.ShapeDtypeStruct((N, N), jnp.bfloat16),
        ],
        compiler_params=pltpu.CompilerParams(
            dimension_semantics=("parallel",),
        ),
        name="msg_mlp",
    )(scal, adj, adj_deriv, idx_emb, embT, wiT, wj)

    out = pl.pallas_call(
        _gnn_kernel,
        in_specs=[
            pl.BlockSpec((N, N), lambda: (0, 0)),
            pl.BlockSpec((N, N), lambda: (0, 0)),
            pl.BlockSpec((N, D), lambda: (0, 0)),
            pl.BlockSpec((N, N), lambda: (0, 0)),
            pl.BlockSpec((L, 5, D, D), lambda: (0, 0, 0, 0)),
            pl.BlockSpec((L, 1, D), lambda: (0, 0, 0)),
        ],
        out_specs=pl.BlockSpec((N, D), lambda: (0, 0)),
        out_shape=jax.ShapeDtypeStruct((N, D), jnp.float32),
        name="gnn_layers",
    )(a, ad, y, t_grad, gnn_W, gnn_b.reshape(L, 1, D))
    return out


# speedup vs baseline: 1.7716x; 1.1144x over previous
import jax
import jax.numpy as jnp
from jax import lax
from jax.experimental import pallas as pl
from jax.experimental.pallas import tpu as pltpu

N = 1024
D = 64
IDX = 64
H = 8
L = 3
BR = 128
NT = N // BR


def _fused_kernel(scal_ref, adj_ref, adjd_ref, t_ref, emb_ref, embblk_ref,
                  w1_ref, y_ref, W_ref, b_ref, o_ref,
                  a_s, ad_s, tg_s, embT_s):
    bf = jnp.bfloat16
    i = pl.program_id(0)

    @pl.when(i == 0)
    def _():
        embT_s[...] = emb_ref[...].T

    for m in range(2):
        A = (adj_ref[...] if m == 0 else adjd_ref[...]).astype(bf)
        out_s = a_s if m == 0 else ad_s
        base = m * 97
        w1i = w1_ref[m, :, 1:1 + IDX]
        w1j = w1_ref[m, :, 1 + IDX:]
        rw = lax.dot_general(embblk_ref[...], w1i, (((1,), (1,)), ((), ())),
                             preferred_element_type=jnp.float32)
        clT = jnp.dot(w1j, embT_s[...], preferred_element_type=jnp.float32)
        b1col = jnp.stack([scal_ref[base + H + k] for k in range(H)]).reshape(H, 1)
        rwb = rw.astype(bf)
        clT2 = (clT + b1col).astype(bf)
        h1 = []
        for k in range(H):
            wa_k = scal_ref[base + k].astype(bf)
            h1.append(jnp.maximum(A * wa_k + rwb[:, k:k + 1] + clT2[k:k + 1, :],
                                  bf(0.0)))
        h2 = []
        for k2 in range(H):
            acc = h1[0] * scal_ref[base + 16 + k2 * H].astype(bf)
            for j in range(1, H):
                acc = acc + h1[j] * scal_ref[base + 16 + k2 * H + j].astype(bf)
            h2.append(jnp.maximum(acc + scal_ref[base + 80 + k2].astype(bf), bf(0.0)))
        out = h2[0] * scal_ref[base + 88].astype(bf)
        for j in range(1, H):
            out = out + h2[j] * scal_ref[base + 88 + j].astype(bf)
        out_s[pl.ds(i * BR, BR), :] = out + scal_ref[base + 96].astype(bf)

    tg_s[pl.ds(i, 1)] = jnp.sum(t_ref[...], axis=0, keepdims=True).reshape(1, 1, N)

    @pl.when(i == NT - 1)
    def _():
        x = y_ref[...]
        dn_t = (((0,), (0,)), ((), ()))
        for l in range(L):
            xb = x.astype(bf)
            p0 = jnp.dot(a_s[...], xb, preferred_element_type=jnp.float32)
            p1 = lax.dot_general(a_s[...], xb, dn_t, preferred_element_type=jnp.float32)
            p2 = jnp.dot(ad_s[...], xb, preferred_element_type=jnp.float32)
            p3 = lax.dot_general(ad_s[...], xb, dn_t, preferred_element_type=jnp.float32)
            x = (jnp.dot(p0, W_ref[l, 0], preferred_element_type=jnp.float32)
                 + jnp.dot(p1, W_ref[l, 1], preferred_element_type=jnp.float32)
                 + jnp.dot(p2, W_ref[l, 2], preferred_element_type=jnp.float32)
                 + jnp.dot(p3, W_ref[l, 3], preferred_element_type=jnp.float32)
                 + jnp.dot(x, W_ref[l, 4], preferred_element_type=jnp.float32)
                 + b_ref[l].reshape(1, D))
            if l < L - 1:
                x = jnp.maximum(x, 0.0)
        ones8 = jnp.ones((NT, D), jnp.float32)
        tgp = tg_s[...].reshape(NT, N)
        tgm = lax.dot_general(tgp, ones8, dn_t, preferred_element_type=jnp.float32)
        o_ref[...] = x * (tgm * (1.0 / N))


def kernel(y, adj, adj_deriv, t_grad, idx_emb, msg_W1, msg_b1, msg_W2, msg_b2,
           msg_W3, msg_b3, gnn_W, gnn_b):
    parts = []
    for m in range(2):
        parts += [msg_W1[m, :, 0], msg_b1[m], msg_W2[m].reshape(-1), msg_b2[m],
                  msg_W3[m, 0], msg_b3[m]]
    scal = jnp.concatenate(parts)

    out = pl.pallas_call(
        _fused_kernel,
        grid=(NT,),
        in_specs=[
            pl.BlockSpec(memory_space=pltpu.SMEM),
            pl.BlockSpec((BR, N), lambda i: (i, 0)),
            pl.BlockSpec((BR, N), lambda i: (i, 0)),
            pl.BlockSpec((BR, N), lambda i: (i, 0)),
            pl.BlockSpec((N, IDX), lambda i: (0, 0)),
            pl.BlockSpec((BR, IDX), lambda i: (i, 0)),
            pl.BlockSpec((2, H, 2 * IDX + 1), lambda i: (0, 0, 0)),
            pl.BlockSpec((N, D), lambda i: (0, 0)),
            pl.BlockSpec((L, 5, D, D), lambda i: (0, 0, 0, 0)),
            pl.BlockSpec((L, D), lambda i: (0, 0)),
        ],
        out_specs=pl.BlockSpec((N, D), lambda i: (0, 0)),
        out_shape=jax.ShapeDtypeStruct((N, D), jnp.float32),
        scratch_shapes=[
            pltpu.VMEM((N, N), jnp.bfloat16),
            pltpu.VMEM((N, N), jnp.bfloat16),
            pltpu.VMEM((NT, 1, N), jnp.float32),
            pltpu.VMEM((IDX, N), jnp.float32),
        ],
        compiler_params=pltpu.CompilerParams(
            dimension_semantics=("arbitrary",),
        ),
        name="fused_msg_gnn",
    )(scal, adj, adj_deriv, t_grad, idx_emb, idx_emb, msg_W1, y, gnn_W, gnn_b)
    return out


# speedup vs baseline: 1.9017x; 1.0735x over previous
import jax
import jax.numpy as jnp
from jax import lax
from jax.experimental import pallas as pl
from jax.experimental.pallas import tpu as pltpu

N = 1024
D = 64
IDX = 64
H = 8
L = 3
BR = 128
NT = N // BR


def _fused_kernel(b1_ref, w2_ref, b2_ref, w3_ref, b3_ref,
                  adj_ref, adjd_ref, t_ref, emb_ref, embblk_ref,
                  w1_ref, y_ref, W_ref, b_ref, o_ref,
                  a_s, ad_s, aT_s, adT_s, tg_s, embT_s):
    bf = jnp.bfloat16
    i = pl.program_id(0)

    @pl.when(i == 0)
    def _():
        embT_s[...] = emb_ref[...].T

    for m in range(2):
        A = (adj_ref[...] if m == 0 else adjd_ref[...]).astype(bf)
        out_s = a_s if m == 0 else ad_s
        outT_s = aT_s if m == 0 else adT_s
        w1i = w1_ref[m, :, 1:1 + IDX]
        w1j = w1_ref[m, :, 1 + IDX:]
        rw = lax.dot_general(embblk_ref[...], w1i, (((1,), (1,)), ((), ())),
                             preferred_element_type=jnp.float32)
        clT = jnp.dot(w1j, embT_s[...], preferred_element_type=jnp.float32)
        b1col = jnp.stack([b1_ref[m, k] for k in range(H)]).reshape(H, 1)
        rwb = rw.astype(bf)
        clT2 = (clT + b1col).astype(bf)
        h1 = []
        for k in range(H):
            wa_k = w1_ref[m, k, 0].astype(bf)
            h1.append(jnp.maximum(A * wa_k + rwb[:, k:k + 1] + clT2[k:k + 1, :],
                                  bf(0.0)))
        h2 = []
        for k2 in range(H):
            acc = h1[0] * w2_ref[m, k2 * H].astype(bf)
            for j in range(1, H):
                acc = acc + h1[j] * w2_ref[m, k2 * H + j].astype(bf)
            h2.append(jnp.maximum(acc + b2_ref[m, k2].astype(bf), bf(0.0)))
        out = h2[0] * w3_ref[m, 0].astype(bf)
        for j in range(1, H):
            out = out + h2[j] * w3_ref[m, j].astype(bf)
        out = out + b3_ref[m, 0].astype(bf)
        out_s[pl.ds(i * BR, BR), :] = out
        outT_s[:, pl.ds(i * BR, BR)] = out.T

    tg_s[pl.ds(i, 1)] = jnp.sum(t_ref[...], axis=0, keepdims=True).reshape(1, 1, N)

    @pl.when(i == NT - 1)
    def _():
        x = y_ref[...]
        for l in range(L):
            xb = x.astype(bf)
            p0 = jnp.dot(a_s[...], xb, preferred_element_type=jnp.float32)
            p1 = jnp.dot(aT_s[...], xb, preferred_element_type=jnp.float32)
            p2 = jnp.dot(ad_s[...], xb, preferred_element_type=jnp.float32)
            p3 = jnp.dot(adT_s[...], xb, preferred_element_type=jnp.float32)
            x = (jnp.dot(p0, W_ref[l, 0], preferred_element_type=jnp.float32)
                 + jnp.dot(p1, W_ref[l, 1], preferred_element_type=jnp.float32)
                 + jnp.dot(p2, W_ref[l, 2], preferred_element_type=jnp.float32)
                 + jnp.dot(p3, W_ref[l, 3], preferred_element_type=jnp.float32)
                 + jnp.dot(x, W_ref[l, 4], preferred_element_type=jnp.float32)
                 + b_ref[l].reshape(1, D))
            if l < L - 1:
                x = jnp.maximum(x, 0.0)
        ones8 = jnp.ones((NT, D), jnp.float32)
        tgp = tg_s[...].reshape(NT, N)
        tgm = lax.dot_general(tgp, ones8, (((0,), (0,)), ((), ())),
                              preferred_element_type=jnp.float32)
        o_ref[...] = x * (tgm * (1.0 / N))


def kernel(y, adj, adj_deriv, t_grad, idx_emb, msg_W1, msg_b1, msg_W2, msg_b2,
           msg_W3, msg_b3, gnn_W, gnn_b):
    smem = pl.BlockSpec(memory_space=pltpu.SMEM)
    out = pl.pallas_call(
        _fused_kernel,
        grid=(NT,),
        in_specs=[
            smem, smem, smem, smem, smem,
            pl.BlockSpec((BR, N), lambda i: (i, 0)),
            pl.BlockSpec((BR, N), lambda i: (i, 0)),
            pl.BlockSpec((BR, N), lambda i: (i, 0)),
            pl.BlockSpec((N, IDX), lambda i: (0, 0)),
            pl.BlockSpec((BR, IDX), lambda i: (i, 0)),
            pl.BlockSpec((2, H, 2 * IDX + 1), lambda i: (0, 0, 0)),
            pl.BlockSpec((N, D), lambda i: (0, 0)),
            pl.BlockSpec((L, 5, D, D), lambda i: (0, 0, 0, 0)),
            pl.BlockSpec((L, D), lambda i: (0, 0)),
        ],
        out_specs=pl.BlockSpec((N, D), lambda i: (0, 0)),
        out_shape=jax.ShapeDtypeStruct((N, D), jnp.float32),
        scratch_shapes=[
            pltpu.VMEM((N, N), jnp.bfloat16),
            pltpu.VMEM((N, N), jnp.bfloat16),
            pltpu.VMEM((N, N), jnp.bfloat16),
            pltpu.VMEM((N, N), jnp.bfloat16),
            pltpu.VMEM((NT, 1, N), jnp.float32),
            pltpu.VMEM((IDX, N), jnp.float32),
        ],
        compiler_params=pltpu.CompilerParams(
            dimension_semantics=("arbitrary",),
        ),
        name="fused_msg_gnn",
    )(msg_b1, msg_W2.reshape(2, H * H), msg_b2, msg_W3.reshape(2, H), msg_b3,
      adj, adj_deriv, t_grad, idx_emb, idx_emb, msg_W1, y, gnn_W, gnn_b)
    return out
